# baseline (speedup 1.0000x reference)
import functools

import jax
import jax.numpy as jnp
from jax import lax
from jax.experimental import pallas as pl
from jax.experimental.pallas import tpu as pltpu
from jax.experimental.pallas import tpu_sc as plsc

N = 10000
E = 320000
D = 128
K = 8
KK = K * K

NC = 2
NS = 16
NW = NC * NS
EPT = E // NW
C = 40
NCHUNK = EPT // C
NPAIR = NCHUNK // 2
NP = 10240
RPT = NP // NS
ZR = 80

_mesh = plsc.VectorSubcoreMesh(core_axis_name="c", subcore_axis_name="s")


def _gather_set():
    return [
        pltpu.VMEM((C, D), jnp.float32),
        pltpu.VMEM((C, D), jnp.float32),
        pltpu.VMEM((C, D), jnp.float32),
        pltpu.VMEM((C, D), jnp.float32),
        pltpu.VMEM((C, KK), jnp.float32),
        pltpu.VMEM((C, KK), jnp.float32),
        pltpu.VMEM((C, KK), jnp.float32),
        pltpu.VMEM((C,), jnp.int32),
        pltpu.VMEM((C,), jnp.int32),
        pltpu.SemaphoreType.DMA,
        pltpu.SemaphoreType.DMA,
    ]


@functools.partial(
    pl.kernel,
    out_type=jax.ShapeDtypeStruct((NC, NP, KK), jnp.float32),
    mesh=_mesh,
    scratch_types=[
        pltpu.VMEM((2 * EPT,), jnp.int32),
        pltpu.VMEM((ZR, KK), jnp.float32),
        pltpu.VMEM_SHARED((NP, KK), jnp.float32),
    ] + _gather_set() + _gather_set(),
    compiler_params=pltpu.CompilerParams(use_tc_tiling_on_sc=False),
)
def _edge_scatter(eflat_hbm, x_hbm, v_hbm, w_hbm, out_hbm,
                  ei_all, zb_v, S_sh, *bufs):
    sets = [bufs[0:11], bufs[11:22]]
    cid = lax.axis_index("c")
    sid = lax.axis_index("s")
    wid = cid * NS + sid

    zero16 = jnp.zeros((16,), jnp.float32)
    lane = lax.iota(jnp.int32, 16)
    bfly = [jnp.bitwise_xor(lane, jnp.int32(1 << k)) for k in range(4)]

    def _allsum(vec):
        for p in bfly:
            vec = vec + vec[p]
        return vec

    def zrow(i, carry):
        for g in range(KK // 16):
            zb_v[i, pl.ds(16 * g, 16)] = zero16
        return carry

    lax.fori_loop(0, ZR, zrow, 0)
    for r in range(RPT // ZR):
        pltpu.sync_copy(zb_v, S_sh.at[pl.ds(sid * RPT + r * ZR, ZR)])

    ebase = wid * EPT
    pltpu.sync_copy(eflat_hbm.at[pl.ds(2 * ebase, 2 * EPT)], ei_all)
    plsc.subcore_barrier()

    evens = (lane % 8) * 2
    odds = evens + 1
    lowmask = lane < 8

    def fill_idx(s, ci):
        ssi_v, sti_v = s[7], s[8]
        off2 = 2 * ci * C
        for q in (0, 16, C - 16):
            a = ei_all[pl.ds(off2 + 2 * q, 16)]
            b = ei_all[pl.ds(off2 + 2 * q + 16, 16)]
            ssi_v[pl.ds(q, 16)] = jnp.where(lowmask, a[evens], b[evens])
            sti_v[pl.ds(q, 16)] = jnp.where(lowmask, a[odds], b[odds])

    def prefetch(s, ci):
        xs_v, xt_v, vs_v, vt_v, w_v, ssi_v, sti_v = s[0], s[1], s[2], s[3], s[4], s[7], s[8]
        gsem = s[9]
        off = ci * C
        pltpu.async_copy(x_hbm.at[ssi_v], xs_v, gsem)
        pltpu.async_copy(x_hbm.at[sti_v], xt_v, gsem)
        pltpu.async_copy(v_hbm.at[ssi_v], vs_v, gsem)
        pltpu.async_copy(v_hbm.at[sti_v], vt_v, gsem)
        pltpu.async_copy(w_hbm.at[pl.ds(ebase + off, C)], w_v, gsem)

    def wait_gathers(s, ci):
        xs_v, xt_v, vs_v, vt_v, w_v, ssi_v = s[0], s[1], s[2], s[3], s[4], s[7]
        gsem = s[9]
        off = ci * C
        pltpu.make_async_copy(x_hbm.at[ssi_v], xs_v, gsem).wait()
        pltpu.make_async_copy(x_hbm.at[ssi_v], xt_v, gsem).wait()
        pltpu.make_async_copy(v_hbm.at[ssi_v], vs_v, gsem).wait()
        pltpu.make_async_copy(v_hbm.at[ssi_v], vt_v, gsem).wait()
        pltpu.make_async_copy(w_hbm.at[pl.ds(ebase + off, C)], w_v, gsem).wait()

    def wait_scatters(s):
        ss_v, st_v, ssi_v, sti_v, ssem = s[5], s[6], s[7], s[8], s[10]
        pltpu.make_async_copy(ss_v, S_sh.at[ssi_v], ssem).wait()
        pltpu.make_async_copy(st_v, S_sh.at[sti_v], ssem).wait()

    def compute(s, ci):
        xs_v, xt_v, vs_v, vt_v, w_v, ss_v, st_v = s[:7]

        @plsc.parallel_loop(0, C, unroll=8)
        def edge_body(i):
            dacc = jnp.zeros((16,), jnp.float32)
            sacc = jnp.zeros((16,), jnp.float32)
            tacc = jnp.zeros((16,), jnp.float32)
            for j in range(D // 16):
                sl = pl.ds(16 * j, 16)
                a = xs_v[i, sl]
                b = xt_v[i, sl]
                dirj = b - a
                dacc = dacc + dirj * dirj
                sacc = sacc + vs_v[i, sl] * dirj
                tacc = tacc + vt_v[i, sl] * dirj
            r = jnp.float32(1.0) / jnp.maximum(_allsum(dacc), jnp.float32(1e-6))
            cs = _allsum(sacc) * r
            ct = _allsum(tacc) * r
            for g in range(KK // 16):
                sl = pl.ds(16 * g, 16)
                wrow = w_v[i, sl]
                ss_v[i, sl] = wrow * cs
                st_v[i, sl] = wrow * ct

    def scatter(s):
        ss_v, st_v, ssi_v, sti_v, ssem = s[5], s[6], s[7], s[8], s[10]
        pltpu.async_copy(ss_v, S_sh.at[ssi_v], ssem, add=True)
        pltpu.async_copy(st_v, S_sh.at[sti_v], ssem, add=True)

    fill_idx(sets[0], 0)
    prefetch(sets[0], 0)

    def pair_body(g, carry):
        @pl.when(g > 0)
        def _():
            wait_scatters(sets[1])

        fill_idx(sets[1], 2 * g + 1)
        prefetch(sets[1], 2 * g + 1)
        wait_gathers(sets[0], 2 * g)
        compute(sets[0], 2 * g)
        scatter(sets[0])

        @pl.when(g < NPAIR - 1)
        def _():
            wait_scatters(sets[0])
            fill_idx(sets[0], 2 * g + 2)
            prefetch(sets[0], 2 * g + 2)

        wait_gathers(sets[1], 2 * g + 1)
        compute(sets[1], 2 * g + 1)
        scatter(sets[1])
        return carry

    lax.fori_loop(0, NPAIR, pair_body, 0)
    wait_scatters(sets[0])
    wait_scatters(sets[1])
    plsc.subcore_barrier()

    for r in range(RPT // ZR):
        pltpu.sync_copy(S_sh.at[pl.ds(sid * RPT + r * ZR, ZR)], zb_v)
        pltpu.sync_copy(zb_v, out_hbm.at[cid, pl.ds(sid * RPT + r * ZR, ZR)])


def _combine_body(p_ref, perm_ref, o_ref):
    s = p_ref[0] + p_ref[1]
    t = jnp.dot(s, perm_ref[...], preferred_element_type=jnp.float32)
    o_ref[...] = 0.5 * (s - t)


def _transpose_perm():
    j = jnp.arange(KK)
    src = K * (j % K) + j // K
    return jnp.zeros((KK, KK), jnp.float32).at[src, j].set(1.0)


def kernel(x, v, edges, omega_params):
    eflat = edges.reshape(2 * E)
    wflat = omega_params.reshape(E, KK)
    partials = _edge_scatter(eflat, x, v, wflat)
    perm = _transpose_perm()
    out = pl.pallas_call(
        _combine_body,
        out_shape=jax.ShapeDtypeStruct((NP, KK), jnp.float32),
    )(partials, perm)
    return out[:N].reshape(N, K, K)


# speedup vs baseline: 1.0218x; 1.0218x over previous
import functools

import jax
import jax.numpy as jnp
from jax import lax
from jax.experimental import pallas as pl
from jax.experimental.pallas import tpu as pltpu
from jax.experimental.pallas import tpu_sc as plsc

N = 10000
E = 320000
D = 128
K = 8
KK = K * K

NC = 2
NS = 16
NW = NC * NS
EPT = E // NW
C = 40
NCHUNK = EPT // C
NPAIR = NCHUNK // 2
EBLK = 2000
NP = 10240
RPT = NP // NS
ZR = 80

_mesh = plsc.VectorSubcoreMesh(core_axis_name="c", subcore_axis_name="s")


def _gather_set():
    return [
        pltpu.VMEM((C, D), jnp.float32),
        pltpu.VMEM((C, D), jnp.float32),
        pltpu.VMEM((C, D), jnp.float32),
        pltpu.VMEM((C, D), jnp.float32),
        pltpu.VMEM((C, KK), jnp.float32),
        pltpu.VMEM((C, KK), jnp.float32),
        pltpu.VMEM((C, KK), jnp.float32),
        pltpu.VMEM((C,), jnp.int32),
        pltpu.VMEM((C,), jnp.int32),
        pltpu.SemaphoreType.DMA,
        pltpu.SemaphoreType.DMA,
    ]


@functools.partial(
    pl.kernel,
    out_type=jax.ShapeDtypeStruct((NC, NP, KK), jnp.float32),
    mesh=_mesh,
    scratch_types=[
        pltpu.VMEM((EPT,), jnp.int32),
        pltpu.VMEM((EPT,), jnp.int32),
        pltpu.VMEM((2 * EBLK,), jnp.int32),
        pltpu.VMEM((ZR, KK), jnp.float32),
        pltpu.VMEM_SHARED((NP, KK), jnp.float32),
    ] + _gather_set() + _gather_set(),
    compiler_params=pltpu.CompilerParams(use_tc_tiling_on_sc=False),
)
def _edge_scatter(eflat_hbm, x_hbm, v_hbm, w_hbm, out_hbm,
                  si_all, ti_all, ebuf, zb_v, S_sh, *bufs):
    sets = [bufs[0:11], bufs[11:22]]
    cid = lax.axis_index("c")
    sid = lax.axis_index("s")
    wid = cid * NS + sid

    zero16 = jnp.zeros((16,), jnp.float32)
    lane = lax.iota(jnp.int32, 16)
    bfly = [jnp.bitwise_xor(lane, jnp.int32(1 << k)) for k in range(4)]

    def _allsum(vec):
        for p in bfly:
            vec = vec + vec[p]
        return vec

    def zrow(i, carry):
        for g in range(KK // 16):
            zb_v[i, pl.ds(16 * g, 16)] = zero16
        return carry

    lax.fori_loop(0, ZR, zrow, 0)
    for r in range(RPT // ZR):
        pltpu.sync_copy(zb_v, S_sh.at[pl.ds(sid * RPT + r * ZR, ZR)])

    ebase = wid * EPT
    evens = (lane % 8) * 2
    odds = evens + 1
    lowmask = lane < 8
    for blk in range(EPT // EBLK):
        pltpu.sync_copy(
            eflat_hbm.at[pl.ds(2 * ebase + 2 * EBLK * blk, 2 * EBLK)], ebuf)

        def dgrp(k, carry, _blk=blk):
            a = ebuf[pl.ds(32 * k, 16)]
            b = ebuf[pl.ds(32 * k + 16, 16)]
            o = pl.ds(EBLK * _blk + 16 * k, 16)
            si_all[o] = jnp.where(lowmask, a[evens], b[evens])
            ti_all[o] = jnp.where(lowmask, a[odds], b[odds])
            return carry

        lax.fori_loop(0, EBLK // 16, dgrp, 0)
    plsc.subcore_barrier()

    def prefetch(s, ci):
        xs_v, xt_v, vs_v, vt_v, w_v = s[0], s[1], s[2], s[3], s[4]
        gsem = s[9]
        off = ci * C
        sis = si_all.at[pl.ds(off, C)]
        tis = ti_all.at[pl.ds(off, C)]
        pltpu.async_copy(x_hbm.at[sis], xs_v, gsem)
        pltpu.async_copy(x_hbm.at[tis], xt_v, gsem)
        pltpu.async_copy(v_hbm.at[sis], vs_v, gsem)
        pltpu.async_copy(v_hbm.at[tis], vt_v, gsem)
        pltpu.async_copy(w_hbm.at[pl.ds(ebase + off, C)], w_v, gsem)

    def wait_gathers(s, ci):
        xs_v, xt_v, vs_v, vt_v, w_v = s[0], s[1], s[2], s[3], s[4]
        gsem = s[9]
        off = ci * C
        sis = si_all.at[pl.ds(off, C)]
        pltpu.make_async_copy(x_hbm.at[sis], xs_v, gsem).wait()
        pltpu.make_async_copy(x_hbm.at[sis], xt_v, gsem).wait()
        pltpu.make_async_copy(v_hbm.at[sis], vs_v, gsem).wait()
        pltpu.make_async_copy(v_hbm.at[sis], vt_v, gsem).wait()
        pltpu.make_async_copy(w_hbm.at[pl.ds(ebase + off, C)], w_v, gsem).wait()

    def wait_scatters(s):
        ss_v, st_v, ssi_v, sti_v, ssem = s[5], s[6], s[7], s[8], s[10]
        pltpu.make_async_copy(ss_v, S_sh.at[ssi_v], ssem).wait()
        pltpu.make_async_copy(st_v, S_sh.at[sti_v], ssem).wait()

    def compute(s, ci):
        xs_v, xt_v, vs_v, vt_v, w_v, ss_v, st_v, ssi_v, sti_v = s[:9]
        off = ci * C
        for q in (0, 16, C - 16):
            ssi_v[pl.ds(q, 16)] = si_all[pl.ds(off + q, 16)]
            sti_v[pl.ds(q, 16)] = ti_all[pl.ds(off + q, 16)]

        @plsc.parallel_loop(0, C, unroll=8)
        def edge_body(i):
            dacc = jnp.zeros((16,), jnp.float32)
            sacc = jnp.zeros((16,), jnp.float32)
            tacc = jnp.zeros((16,), jnp.float32)
            for j in range(D // 16):
                sl = pl.ds(16 * j, 16)
                a = xs_v[i, sl]
                b = xt_v[i, sl]
                dirj = b - a
                dacc = dacc + dirj * dirj
                sacc = sacc + vs_v[i, sl] * dirj
                tacc = tacc + vt_v[i, sl] * dirj
            r = jnp.float32(1.0) / jnp.maximum(_allsum(dacc), jnp.float32(1e-6))
            cs = _allsum(sacc) * r
            ct = _allsum(tacc) * r
            for g in range(KK // 16):
                sl = pl.ds(16 * g, 16)
                wrow = w_v[i, sl]
                ss_v[i, sl] = wrow * cs
                st_v[i, sl] = wrow * ct

    def scatter(s):
        ss_v, st_v, ssi_v, sti_v, ssem = s[5], s[6], s[7], s[8], s[10]
        pltpu.async_copy(ss_v, S_sh.at[ssi_v], ssem, add=True)
        pltpu.async_copy(st_v, S_sh.at[sti_v], ssem, add=True)

    prefetch(sets[0], 0)

    def pair_body(g, carry):
        prefetch(sets[1], 2 * g + 1)
        wait_gathers(sets[0], 2 * g)

        @pl.when(g > 0)
        def _():
            wait_scatters(sets[0])

        compute(sets[0], 2 * g)
        scatter(sets[0])

        @pl.when(g < NPAIR - 1)
        def _():
            prefetch(sets[0], 2 * g + 2)

        wait_gathers(sets[1], 2 * g + 1)

        @pl.when(g > 0)
        def _():
            wait_scatters(sets[1])

        compute(sets[1], 2 * g + 1)
        scatter(sets[1])
        return carry

    lax.fori_loop(0, NPAIR, pair_body, 0)
    wait_scatters(sets[0])
    wait_scatters(sets[1])
    plsc.subcore_barrier()

    for r in range(RPT // ZR):
        pltpu.sync_copy(S_sh.at[pl.ds(sid * RPT + r * ZR, ZR)], zb_v)
        pltpu.sync_copy(zb_v, out_hbm.at[cid, pl.ds(sid * RPT + r * ZR, ZR)])


def _combine_body(p_ref, perm_ref, o_ref):
    s = p_ref[0] + p_ref[1]
    t = jnp.dot(s, perm_ref[...], preferred_element_type=jnp.float32)
    o_ref[...] = 0.5 * (s - t)


def _transpose_perm():
    j = jnp.arange(KK)
    src = K * (j % K) + j // K
    return jnp.zeros((KK, KK), jnp.float32).at[src, j].set(1.0)


def kernel(x, v, edges, omega_params):
    eflat = edges.reshape(2 * E)
    wflat = omega_params.reshape(E, KK)
    partials = _edge_scatter(eflat, x, v, wflat)
    perm = _transpose_perm()
    out = pl.pallas_call(
        _combine_body,
        out_shape=jax.ShapeDtypeStruct((NP, KK), jnp.float32),
    )(partials, perm)
    return out[:N].reshape(N, K, K)


# speedup vs baseline: 1.3493x; 1.3204x over previous
import functools

import jax
import jax.numpy as jnp
from jax import lax
from jax.experimental import pallas as pl
from jax.experimental.pallas import tpu as pltpu
from jax.experimental.pallas import tpu_sc as plsc

N = 10000
E = 320000
D = 128
K = 8
KK = K * K

NC = 2
NS = 16
NW = NC * NS
EPT = E // NW
C = 40
NCHUNK = EPT // C
NPAIR = NCHUNK // 2
EBLK = 2000
NP = 10240
RPT = NP // NS
ZR = 80

_mesh = plsc.VectorSubcoreMesh(core_axis_name="c", subcore_axis_name="s")


def _gather_set():
    return [
        pltpu.VMEM((C, D), jnp.float32),
        pltpu.VMEM((C, D), jnp.float32),
        pltpu.VMEM((C, D), jnp.float32),
        pltpu.VMEM((C, D), jnp.float32),
        pltpu.VMEM((C, KK), jnp.float32),
        pltpu.VMEM((C, KK), jnp.float32),
        pltpu.VMEM((C, KK), jnp.float32),
        pltpu.VMEM((C,), jnp.int32),
        pltpu.VMEM((C,), jnp.int32),
        pltpu.SemaphoreType.DMA,
        pltpu.SemaphoreType.DMA,
    ]


@functools.partial(
    pl.kernel,
    out_type=jax.ShapeDtypeStruct((NC, NP, KK), jnp.float32),
    mesh=_mesh,
    scratch_types=[
        pltpu.VMEM((EPT,), jnp.int32),
        pltpu.VMEM((EPT,), jnp.int32),
        pltpu.VMEM((ZR, KK), jnp.float32),
        pltpu.VMEM_SHARED((NP, KK), jnp.float32),
    ] + _gather_set() + _gather_set(),
    compiler_params=pltpu.CompilerParams(use_tc_tiling_on_sc=False),
)
def _edge_scatter(et_hbm, x_hbm, v_hbm, w_hbm, out_hbm,
                  si_all, ti_all, zb_v, S_sh, *bufs):
    sets = [bufs[0:11], bufs[11:22]]
    cid = lax.axis_index("c")
    sid = lax.axis_index("s")
    wid = cid * NS + sid

    zero16 = jnp.zeros((16,), jnp.float32)
    lane = lax.iota(jnp.int32, 16)
    bfly = [jnp.bitwise_xor(lane, jnp.int32(1 << k)) for k in range(4)]

    def _allsum(vec):
        for p in bfly:
            vec = vec + vec[p]
        return vec

    def zrow(i, carry):
        for g in range(KK // 16):
            zb_v[i, pl.ds(16 * g, 16)] = zero16
        return carry

    lax.fori_loop(0, ZR, zrow, 0)
    for r in range(RPT // ZR):
        pltpu.sync_copy(zb_v, S_sh.at[pl.ds(sid * RPT + r * ZR, ZR)])

    ebase = wid * EPT
    pltpu.sync_copy(et_hbm.at[0, pl.ds(ebase, EPT)], si_all)
    pltpu.sync_copy(et_hbm.at[1, pl.ds(ebase, EPT)], ti_all)
    plsc.subcore_barrier()

    def prefetch(s, ci):
        xs_v, xt_v, vs_v, vt_v, w_v = s[0], s[1], s[2], s[3], s[4]
        gsem = s[9]
        off = ci * C
        sis = si_all.at[pl.ds(off, C)]
        tis = ti_all.at[pl.ds(off, C)]
        pltpu.async_copy(x_hbm.at[sis], xs_v, gsem)
        pltpu.async_copy(x_hbm.at[tis], xt_v, gsem)
        pltpu.async_copy(v_hbm.at[sis], vs_v, gsem)
        pltpu.async_copy(v_hbm.at[tis], vt_v, gsem)
        pltpu.async_copy(w_hbm.at[pl.ds(ebase + off, C)], w_v, gsem)

    def wait_gathers(s, ci):
        xs_v, xt_v, vs_v, vt_v, w_v = s[0], s[1], s[2], s[3], s[4]
        gsem = s[9]
        off = ci * C
        sis = si_all.at[pl.ds(off, C)]
        pltpu.make_async_copy(x_hbm.at[sis], xs_v, gsem).wait()
        pltpu.make_async_copy(x_hbm.at[sis], xt_v, gsem).wait()
        pltpu.make_async_copy(v_hbm.at[sis], vs_v, gsem).wait()
        pltpu.make_async_copy(v_hbm.at[sis], vt_v, gsem).wait()
        pltpu.make_async_copy(w_hbm.at[pl.ds(ebase + off, C)], w_v, gsem).wait()

    def wait_scatters(s):
        ss_v, st_v, ssi_v, sti_v, ssem = s[5], s[6], s[7], s[8], s[10]
        pltpu.make_async_copy(ss_v, S_sh.at[ssi_v], ssem).wait()
        pltpu.make_async_copy(st_v, S_sh.at[sti_v], ssem).wait()

    def compute(s, ci):
        xs_v, xt_v, vs_v, vt_v, w_v, ss_v, st_v, ssi_v, sti_v = s[:9]
        off = ci * C
        for q in (0, 16, C - 16):
            ssi_v[pl.ds(q, 16)] = si_all[pl.ds(off + q, 16)]
            sti_v[pl.ds(q, 16)] = ti_all[pl.ds(off + q, 16)]

        @plsc.parallel_loop(0, C, unroll=8)
        def edge_body(i):
            dacc = jnp.zeros((16,), jnp.float32)
            sacc = jnp.zeros((16,), jnp.float32)
            tacc = jnp.zeros((16,), jnp.float32)
            for j in range(D // 16):
                sl = pl.ds(16 * j, 16)
                a = xs_v[i, sl]
                b = xt_v[i, sl]
                dirj = b - a
                dacc = dacc + dirj * dirj
                sacc = sacc + vs_v[i, sl] * dirj
                tacc = tacc + vt_v[i, sl] * dirj
            r = jnp.float32(1.0) / jnp.maximum(_allsum(dacc), jnp.float32(1e-6))
            cs = _allsum(sacc) * r
            ct = _allsum(tacc) * r
            for g in range(KK // 16):
                sl = pl.ds(16 * g, 16)
                wrow = w_v[i, sl]
                ss_v[i, sl] = wrow * cs
                st_v[i, sl] = wrow * ct

    def scatter(s):
        ss_v, st_v, ssi_v, sti_v, ssem = s[5], s[6], s[7], s[8], s[10]
        pltpu.async_copy(ss_v, S_sh.at[ssi_v], ssem, add=True)
        pltpu.async_copy(st_v, S_sh.at[sti_v], ssem, add=True)

    prefetch(sets[0], 0)

    def pair_body(g, carry):
        prefetch(sets[1], 2 * g + 1)
        wait_gathers(sets[0], 2 * g)

        @pl.when(g > 0)
        def _():
            wait_scatters(sets[0])

        compute(sets[0], 2 * g)
        scatter(sets[0])

        @pl.when(g < NPAIR - 1)
        def _():
            prefetch(sets[0], 2 * g + 2)

        wait_gathers(sets[1], 2 * g + 1)

        @pl.when(g > 0)
        def _():
            wait_scatters(sets[1])

        compute(sets[1], 2 * g + 1)
        scatter(sets[1])
        return carry

    lax.fori_loop(0, NPAIR, pair_body, 0)
    wait_scatters(sets[0])
    wait_scatters(sets[1])
    plsc.subcore_barrier()

    pltpu.sync_copy(S_sh.at[pl.ds(sid * RPT, RPT)],
                    out_hbm.at[cid, pl.ds(sid * RPT, RPT)])


def _combine_body(p_ref, perm_ref, o_ref):
    s = p_ref[0] + p_ref[1]
    t = jnp.dot(s, perm_ref[...], preferred_element_type=jnp.float32)
    o_ref[...] = 0.5 * (s - t)


def _transpose_perm():
    j = jnp.arange(KK)
    src = K * (j % K) + j // K
    return jnp.zeros((KK, KK), jnp.float32).at[src, j].set(1.0)


def kernel(x, v, edges, omega_params):
    et = edges.T
    wflat = omega_params.reshape(E, KK)
    partials = _edge_scatter(et, x, v, wflat)
    perm = _transpose_perm()
    out = pl.pallas_call(
        _combine_body,
        out_shape=jax.ShapeDtypeStruct((NP, KK), jnp.float32),
    )(partials, perm)
    return out[:N].reshape(N, K, K)
